# baseline (speedup 1.0000x reference)
import functools

import jax
import jax.numpy as jnp
from jax import lax
from jax.experimental import pallas as pl
from jax.experimental.pallas import tpu as pltpu
from jax.experimental.pallas import tpu_sc as plsc

N_TASKS = 1000000
B = 16384
D = 64
NC = 2
NS = 16
NW = NC * NS
BPW = B // NW
CH = 128
NCHD = BPW // CH

_mesh = plsc.VectorSubcoreMesh(core_axis_name="c", subcore_axis_name="s")


@functools.partial(
    pl.kernel,
    out_type=jax.ShapeDtypeStruct((D, B), jnp.float32),
    mesh=_mesh,
    scratch_types=[
        pltpu.VMEM((BPW,), jnp.int32),
        pltpu.VMEM((D * BPW,), jnp.int32),
        pltpu.VMEM((D, BPW), jnp.float32),
        pltpu.SemaphoreType.DMA,
    ],
    compiler_params=pltpu.CompilerParams(use_tc_tiling_on_sc=False),
)
def _gather_kernel(idx_hbm, flat_hbm, out_t_hbm, idx_v, addr_v, slab_v, sem):
    wid = lax.axis_index("s") * NC + lax.axis_index("c")
    base = wid * BPW
    pltpu.sync_copy(idx_hbm.at[pl.ds(base, BPW)], idx_v)

    def build(d, carry):
        for j in range(BPW // 16):
            idx16 = idx_v[pl.ds(j * 16, 16)]
            addr_v[pl.ds(d * BPW + j * 16, 16)] = idx16 + d * N_TASKS
        return carry

    lax.fori_loop(0, D, build, 0)

    copies = []
    for r in range(D * NCHD):
        copies.append(
            pltpu.async_copy(
                flat_hbm.at[addr_v.at[pl.ds(r * CH, CH)]],
                slab_v.at[r // NCHD, pl.ds((r % NCHD) * CH, CH)],
                sem,
            )
        )
    for c in copies:
        c.wait()

    pltpu.sync_copy(slab_v, out_t_hbm.at[:, pl.ds(base, BPW)])


def kernel(task_id, embedding_weight):
    idx = task_id.astype(jnp.int32)
    flat = embedding_weight.T.reshape(D * N_TASKS)
    out_t = _gather_kernel(idx, flat)
    return out_t.T


# speedup vs baseline: 15.9009x; 15.9009x over previous
import functools

import jax
import jax.numpy as jnp
from jax import lax
from jax.experimental import pallas as pl
from jax.experimental.pallas import tpu as pltpu
from jax.experimental.pallas import tpu_sc as plsc

N_TASKS = 1000000
B = 16384
D = 64
NC = 2
NS = 16
NW = NC * NS
BPW = B // NW
QW = 4
SUB = 128
NSUB = BPW // SUB

_mesh = plsc.VectorSubcoreMesh(core_axis_name="c", subcore_axis_name="s")


@functools.partial(
    pl.kernel,
    out_type=jax.ShapeDtypeStruct((D, B), jnp.float32),
    mesh=_mesh,
    scratch_types=[
        pltpu.VMEM((BPW,), jnp.int32),
        pltpu.VMEM((QW, D, 128), jnp.float32),
        pltpu.VMEM((D, SUB), jnp.float32),
        pltpu.SemaphoreType.DMA,
    ],
    compiler_params=pltpu.CompilerParams(needs_layout_passes=False),
)
def _gather_kernel(idx_hbm, table_t_hbm, out_t_hbm, idx_v, slabs_v, osub_v, sem):
    wid = lax.axis_index("s") * NC + lax.axis_index("c")
    base = wid * BPW
    pltpu.sync_copy(idx_hbm.at[pl.ds(base, BPW)], idx_v)

    iota16 = lax.iota(jnp.int32, 16)
    rows16 = [iota16 + 16 * q for q in range(4)]

    for sub in range(NSUB):
        def group(g, carry, sub=sub):
            idx16 = idx_v[pl.ds(sub * SUB + g * 16, 16)]
            for p in range(4):
                copies = []
                scalars = []
                for q4 in range(4):
                    i = idx16[p * 4 + q4]
                    off = pl.multiple_of((i >> 7) * 128, 128)
                    scalars.append(i & 127)
                    copies.append(
                        pltpu.async_copy(
                            table_t_hbm.at[:, pl.ds(off, 128)],
                            slabs_v.at[q4],
                            sem,
                        )
                    )
                for c in copies:
                    c.wait()
                for q4 in range(4):
                    lanes = jnp.broadcast_to(scalars[q4], (16,))
                    col = jnp.broadcast_to(g * 16 + p * 4 + q4, (16,))
                    for q in range(4):
                        vals = plsc.load_gather(
                            slabs_v.at[q4], [rows16[q], lanes]
                        )
                        plsc.store_scatter(osub_v, [rows16[q], col], vals)
            return carry

        lax.fori_loop(0, SUB // 16, group, 0)
        pltpu.sync_copy(
            osub_v, out_t_hbm.at[:, pl.ds(base + sub * SUB, SUB)]
        )


def kernel(task_id, embedding_weight):
    idx = task_id.astype(jnp.int32)
    out_t = _gather_kernel(idx, embedding_weight.T)
    return out_t.T


# speedup vs baseline: 22.2775x; 1.4010x over previous
import functools

import jax
import jax.numpy as jnp
from jax import lax
from jax.experimental import pallas as pl
from jax.experimental.pallas import tpu as pltpu
from jax.experimental.pallas import tpu_sc as plsc

N_TASKS = 1000000
B = 16384
D = 64
NC = 2
NS = 16
NW = NC * NS
BPW = B // NW
SUB = 128
NSUB = BPW // SUB

_mesh = plsc.VectorSubcoreMesh(core_axis_name="c", subcore_axis_name="s")


@functools.partial(
    pl.kernel,
    out_type=jax.ShapeDtypeStruct((D, B), jnp.float32),
    mesh=_mesh,
    scratch_types=[
        pltpu.VMEM((BPW,), jnp.int32),
        pltpu.VMEM((8, D, 128), jnp.float32),
        pltpu.VMEM((D, SUB), jnp.float32),
        pltpu.SemaphoreType.DMA,
    ],
    compiler_params=pltpu.CompilerParams(needs_layout_passes=False),
)
def _gather_kernel(idx_hbm, table_t_hbm, out_t_hbm, idx_v, slabs_v, osub_v, sem):
    wid = lax.axis_index("s") * NC + lax.axis_index("c")
    base = wid * BPW
    pltpu.sync_copy(idx_hbm.at[pl.ds(base, BPW)], idx_v)

    iota16 = lax.iota(jnp.int32, 16)
    rows16 = [iota16 + 16 * q for q in range(4)]

    def fire(idx16, p):
        copies, lanes = [], []
        for q4 in range(4):
            i = idx16[p * 4 + q4]
            off = pl.multiple_of((i >> 7) * 128, 128)
            lanes.append(i & 127)
            copies.append(
                pltpu.async_copy(
                    table_t_hbm.at[:, pl.ds(off, 128)],
                    slabs_v.at[(p % 2) * 4 + q4],
                    sem,
                )
            )
        return copies, lanes

    def extract(copies, lanes, p, col0):
        for c in copies:
            c.wait()
        for q4 in range(4):
            lane16 = jnp.broadcast_to(lanes[q4], (16,))
            col = jnp.broadcast_to(col0 + p * 4 + q4, (16,))
            for q in range(4):
                vals = plsc.load_gather(
                    slabs_v.at[(p % 2) * 4 + q4], [rows16[q], lane16]
                )
                plsc.store_scatter(osub_v, [rows16[q], col], vals)

    for sub in range(NSUB):
        def group(g, carry, sub=sub):
            idx16 = idx_v[pl.ds(sub * SUB + g * 16, 16)]
            col0 = g * 16
            pend = fire(idx16, 0)
            for p in range(3):
                nxt = fire(idx16, p + 1)
                extract(*pend, p, col0)
                pend = nxt
            extract(*pend, 3, col0)
            return carry

        lax.fori_loop(0, SUB // 16, group, 0)
        pltpu.sync_copy(
            osub_v, out_t_hbm.at[:, pl.ds(base + sub * SUB, SUB)]
        )


def kernel(task_id, embedding_weight):
    idx = task_id.astype(jnp.int32)
    out_t = _gather_kernel(idx, embedding_weight.T)
    return out_t.T


# speedup vs baseline: 25.5488x; 1.1468x over previous
import functools

import jax
import jax.numpy as jnp
from jax import lax
from jax.experimental import pallas as pl
from jax.experimental.pallas import tpu as pltpu
from jax.experimental.pallas import tpu_sc as plsc

N_TASKS = 1000000
B = 16384
D = 64
NC = 2
NS = 16
NW = NC * NS
BPW = B // NW
NG = BPW // 16

_mesh = plsc.VectorSubcoreMesh(core_axis_name="c", subcore_axis_name="s")


@functools.partial(
    pl.kernel,
    out_type=jax.ShapeDtypeStruct((D, B), jnp.float32),
    mesh=_mesh,
    scratch_types=[
        pltpu.VMEM((BPW,), jnp.int32),
        pltpu.VMEM((8, D, 128), jnp.float32),
        pltpu.VMEM((D, BPW), jnp.float32),
        pltpu.SemaphoreType.DMA,
    ],
    compiler_params=pltpu.CompilerParams(needs_layout_passes=False),
)
def _gather_kernel(idx_hbm, table_t_hbm, out_t_hbm, idx_v, slabs_v, oslab_v, sem):
    wid = lax.axis_index("s") * NC + lax.axis_index("c")
    base = wid * BPW
    pltpu.sync_copy(idx_hbm.at[pl.ds(base, BPW)], idx_v)

    iota16 = lax.iota(jnp.int32, 16)
    rows16 = [iota16 + 16 * q for q in range(4)]

    def bank(p):
        return (p % 2) * 4

    def fire(idx16, p):
        copies, lanes = [], []
        for q4 in range(4):
            i = idx16[p * 4 + q4]
            off = pl.multiple_of((i >> 7) * 128, 128)
            lanes.append(i & 127)
            copies.append(
                pltpu.async_copy(
                    table_t_hbm.at[:, pl.ds(off, 128)],
                    slabs_v.at[bank(p) + q4],
                    sem,
                )
            )
        return copies, lanes

    def extract(lanes, p, col0):
        for q4 in range(4):
            lane16 = jnp.broadcast_to(lanes[q4], (16,))
            col = jnp.broadcast_to(col0 + p * 4 + q4, (16,))
            for q in range(4):
                vals = plsc.load_gather(
                    slabs_v.at[bank(p) + q4], [rows16[q], lane16]
                )
                plsc.store_scatter(oslab_v, [rows16[q], col], vals)

    def drain_extract_q3(g_prev):
        idx16p = idx_v[pl.ds(g_prev * 16, 16)]
        lanes = [idx16p[12 + q4] & 127 for q4 in range(4)]
        for q4 in range(4):
            pltpu.make_async_copy(
                table_t_hbm.at[:, pl.ds(0, 128)], slabs_v.at[4 + q4], sem
            ).wait()
        extract(lanes, 3, g_prev * 16)

    def group(g, carry):
        idx16 = idx_v[pl.ds(g * 16, 16)]
        col0 = g * 16
        c0, l0 = fire(idx16, 0)

        @pl.when(g > 0)
        def _():
            drain_extract_q3(g - 1)

        c1, l1 = fire(idx16, 1)
        for c in c0:
            c.wait()
        extract(l0, 0, col0)
        c2, l2 = fire(idx16, 2)
        for c in c1:
            c.wait()
        extract(l1, 1, col0)
        _c3, _l3 = fire(idx16, 3)
        for c in c2:
            c.wait()
        extract(l2, 2, col0)
        return carry

    lax.fori_loop(0, NG, group, 0)
    drain_extract_q3(NG - 1)
    pltpu.sync_copy(oslab_v, out_t_hbm.at[:, pl.ds(base, BPW)])


def kernel(task_id, embedding_weight):
    idx = task_id.astype(jnp.int32)
    out_t = _gather_kernel(idx, embedding_weight.T)
    return out_t.T
